# baseline (speedup 1.0000x reference)
import functools

import jax
import jax.numpy as jnp
from jax import lax
from jax.experimental import pallas as pl
from jax.experimental.pallas import tpu as pltpu
from jax.experimental.pallas import tpu_sc as plsc

B, T, C, H, W = 4, 32, 3, 224, 224
S = max(1, T // 4)
ROW = C * H * W
NCHUNK = 2
CH = ROW // NCHUNK

_NC = 2
_NS = 16
_NW = _NC * _NS

_mesh = plsc.VectorSubcoreMesh(core_axis_name="c", subcore_axis_name="s")


@functools.partial(
    pl.kernel,
    out_type=jax.ShapeDtypeStruct((B * S, NCHUNK, CH), jnp.float32),
    mesh=_mesh,
    scratch_types=[
        pltpu.VMEM((CH,), jnp.float32),
        pltpu.SemaphoreType.DMA,
    ],
)
def _slow_gather(frames_hbm, out_hbm, buf, sem):
    wid = lax.axis_index("s") * _NC + lax.axis_index("c")
    b = wid // S
    s = wid % S
    src = b * T + (s * (T - 1)) // (S - 1)
    for c in range(NCHUNK):
        pltpu.async_copy(frames_hbm.at[src, c], buf, sem).wait()
        pltpu.async_copy(buf, out_hbm.at[wid, c], sem).wait()


def kernel(frames):
    flat = frames.reshape(B * T, NCHUNK, CH)
    slow = _slow_gather(flat).reshape(B, S, C, H, W)
    return (slow, frames)


# speedup vs baseline: 2.8997x; 2.8997x over previous
import functools

import jax
import jax.numpy as jnp
from jax import lax
from jax.experimental import pallas as pl
from jax.experimental.pallas import tpu as pltpu
from jax.experimental.pallas import tpu_sc as plsc

B, T, C, H, W = 4, 32, 3, 224, 224
S = max(1, T // 4)
ROW = C * H * W
NCHUNK = 2
CH = ROW // NCHUNK

_NC = 2
_NS = 16
_NW = _NC * _NS

_mesh = plsc.VectorSubcoreMesh(core_axis_name="c", subcore_axis_name="s")


@functools.partial(
    pl.kernel,
    out_type=jax.ShapeDtypeStruct((B, S, C, H, W), jnp.float32),
    mesh=_mesh,
    scratch_types=[
        pltpu.VMEM((H, W), jnp.float32),
        pltpu.SemaphoreType.DMA,
    ],
)
def _slow_gather(frames_hbm, out_hbm, buf, sem):
    wid = lax.axis_index("s") * _NC + lax.axis_index("c")
    b = wid // S
    s = wid % S
    src_t = (s * (T - 1)) // (S - 1)
    for c in range(C):
        pltpu.async_copy(frames_hbm.at[b, src_t, c], buf, sem).wait()
        pltpu.async_copy(buf, out_hbm.at[b, s, c], sem).wait()


def kernel(frames):
    return (_slow_gather(frames), frames)


# speedup vs baseline: 2.9200x; 1.0070x over previous
import functools

import jax
import jax.numpy as jnp
from jax import lax
from jax.experimental import pallas as pl
from jax.experimental.pallas import tpu as pltpu
from jax.experimental.pallas import tpu_sc as plsc

B, T, C, H, W = 4, 32, 3, 224, 224
S = max(1, T // 4)
ROW = C * H * W
NCHUNK = 2
CH = ROW // NCHUNK

_NC = 2
_NS = 16
_NW = _NC * _NS

_mesh = plsc.VectorSubcoreMesh(core_axis_name="c", subcore_axis_name="s")


@functools.partial(
    pl.kernel,
    out_type=jax.ShapeDtypeStruct((B, S, C, H, W), jnp.float32),
    mesh=_mesh,
    scratch_types=[
        pltpu.VMEM((H, W), jnp.float32),
        pltpu.VMEM((H, W), jnp.float32),
        pltpu.SemaphoreType.DMA,
        pltpu.SemaphoreType.DMA,
        pltpu.SemaphoreType.DMA,
        pltpu.SemaphoreType.DMA,
    ],
)
def _slow_gather(frames_hbm, out_hbm, buf0, buf1, si0, si1, so0, so1):
    wid = lax.axis_index("s") * _NC + lax.axis_index("c")
    b = wid // S
    s = wid % S
    src_t = (s * (T - 1)) // (S - 1)
    in0 = pltpu.async_copy(frames_hbm.at[b, src_t, 0], buf0, si0)
    in1 = pltpu.async_copy(frames_hbm.at[b, src_t, 1], buf1, si1)
    in0.wait()
    out0 = pltpu.async_copy(buf0, out_hbm.at[b, s, 0], so0)
    in1.wait()
    out1 = pltpu.async_copy(buf1, out_hbm.at[b, s, 1], so1)
    out0.wait()
    in2 = pltpu.async_copy(frames_hbm.at[b, src_t, 2], buf0, si0)
    in2.wait()
    out2 = pltpu.async_copy(buf0, out_hbm.at[b, s, 2], so0)
    out1.wait()
    out2.wait()


def kernel(frames):
    return (_slow_gather(frames), frames)


# speedup vs baseline: 2.9743x; 1.0186x over previous
import functools

import jax
import jax.numpy as jnp
from jax import lax
from jax.experimental import pallas as pl
from jax.experimental.pallas import tpu as pltpu
from jax.experimental.pallas import tpu_sc as plsc

B, T, C, H, W = 4, 32, 3, 224, 224
S = max(1, T // 4)
ROW = C * H * W
NCHUNK = 2
CH = ROW // NCHUNK

_NC = 2
_NS = 16
_NW = _NC * _NS

_mesh = plsc.VectorSubcoreMesh(core_axis_name="c", subcore_axis_name="s")


@functools.partial(
    pl.kernel,
    out_type=jax.ShapeDtypeStruct((B, S, C, H, W), jnp.float32),
    mesh=_mesh,
    scratch_types=[
        pltpu.VMEM((H, W), jnp.float32),
        pltpu.VMEM((H, W), jnp.float32),
        pltpu.SemaphoreType.DMA,
        pltpu.SemaphoreType.DMA,
        pltpu.SemaphoreType.DMA,
        pltpu.SemaphoreType.DMA,
    ],
)
def _slow_gather(frames_hbm, out_hbm, buf0, buf1, si0, si1, so0, so1):
    wid = lax.axis_index("s") * _NC + lax.axis_index("c")
    b = wid // S
    s = wid % S
    src_t = (s * (T - 1)) // (S - 1)
    in0 = pltpu.async_copy(frames_hbm.at[b, src_t, 0], buf0, si0)
    in1 = pltpu.async_copy(frames_hbm.at[b, src_t, 1], buf1, si1)
    in0.wait()
    out0 = pltpu.async_copy(buf0, out_hbm.at[b, s, 0], so0)
    in1.wait()
    out1 = pltpu.async_copy(buf1, out_hbm.at[b, s, 1], so1)
    out0.wait()
    in2 = pltpu.async_copy(frames_hbm.at[b, src_t, 2], buf0, si0)
    in2.wait()
    out2 = pltpu.async_copy(buf0, out_hbm.at[b, s, 2], so0)
    out1.wait()
    out2.wait()


def _fast_copy_body(x_ref, o_ref):
    o_ref[...] = x_ref[...]


_TBLK = 4


def _fast_copy(frames):
    return pl.pallas_call(
        _fast_copy_body,
        grid=(B, T // _TBLK),
        in_specs=[pl.BlockSpec((1, _TBLK, C, H, W), lambda i, j: (i, j, 0, 0, 0))],
        out_specs=pl.BlockSpec((1, _TBLK, C, H, W), lambda i, j: (i, j, 0, 0, 0)),
        out_shape=jax.ShapeDtypeStruct((B, T, C, H, W), jnp.float32),
    )(frames)


def kernel(frames):
    return (_slow_gather(frames), _fast_copy(frames))


# speedup vs baseline: 3.0549x; 1.0271x over previous
import functools

import jax
import jax.numpy as jnp
from jax import lax
from jax.experimental import pallas as pl
from jax.experimental.pallas import tpu as pltpu
from jax.experimental.pallas import tpu_sc as plsc

B, T, C, H, W = 4, 32, 3, 224, 224
S = max(1, T // 4)
ROW = C * H * W
NCHUNK = 2
CH = ROW // NCHUNK

_NC = 2
_NS = 16
_NW = _NC * _NS

_mesh = plsc.VectorSubcoreMesh(core_axis_name="c", subcore_axis_name="s")


@functools.partial(
    pl.kernel,
    out_type=jax.ShapeDtypeStruct((B, S, C, H, W), jnp.float32),
    mesh=_mesh,
    scratch_types=[
        pltpu.VMEM((H, W), jnp.float32),
        pltpu.VMEM((H, W), jnp.float32),
        pltpu.SemaphoreType.DMA,
        pltpu.SemaphoreType.DMA,
        pltpu.SemaphoreType.DMA,
        pltpu.SemaphoreType.DMA,
    ],
)
def _slow_gather(frames_hbm, out_hbm, buf0, buf1, si0, si1, so0, so1):
    wid = lax.axis_index("s") * _NC + lax.axis_index("c")
    b = wid // S
    s = wid % S
    src_t = (s * (T - 1)) // (S - 1)
    in0 = pltpu.async_copy(frames_hbm.at[b, src_t, 0], buf0, si0)
    in1 = pltpu.async_copy(frames_hbm.at[b, src_t, 1], buf1, si1)
    in0.wait()
    out0 = pltpu.async_copy(buf0, out_hbm.at[b, s, 0], so0)
    in1.wait()
    out1 = pltpu.async_copy(buf1, out_hbm.at[b, s, 1], so1)
    out0.wait()
    in2 = pltpu.async_copy(frames_hbm.at[b, src_t, 2], buf0, si0)
    in2.wait()
    out2 = pltpu.async_copy(buf0, out_hbm.at[b, s, 2], so0)
    out1.wait()
    out2.wait()


def _fast_copy_body(x_ref, o_ref):
    o_ref[...] = x_ref[...]


_TBLK = 8


def _fast_copy(frames):
    return pl.pallas_call(
        _fast_copy_body,
        grid=(B, T // _TBLK),
        in_specs=[pl.BlockSpec((1, _TBLK, C, H, W), lambda i, j: (i, j, 0, 0, 0))],
        out_specs=pl.BlockSpec((1, _TBLK, C, H, W), lambda i, j: (i, j, 0, 0, 0)),
        out_shape=jax.ShapeDtypeStruct((B, T, C, H, W), jnp.float32),
    )(frames)


def kernel(frames):
    return (_slow_gather(frames), _fast_copy(frames))


# speedup vs baseline: 3.1201x; 1.0213x over previous
import functools

import jax
import jax.numpy as jnp
from jax import lax
from jax.experimental import pallas as pl
from jax.experimental.pallas import tpu as pltpu
from jax.experimental.pallas import tpu_sc as plsc

B, T, C, H, W = 4, 32, 3, 224, 224
S = max(1, T // 4)
ROW = C * H * W
NCHUNK = 2
CH = ROW // NCHUNK

_NC = 2
_NS = 16
_NW = _NC * _NS

_mesh = plsc.VectorSubcoreMesh(core_axis_name="c", subcore_axis_name="s")


@functools.partial(
    pl.kernel,
    out_type=jax.ShapeDtypeStruct((B, S, C, H, W), jnp.float32),
    mesh=_mesh,
    scratch_types=[
        pltpu.VMEM((H, W), jnp.float32),
        pltpu.VMEM((H, W), jnp.float32),
        pltpu.SemaphoreType.DMA,
        pltpu.SemaphoreType.DMA,
        pltpu.SemaphoreType.DMA,
        pltpu.SemaphoreType.DMA,
    ],
)
def _slow_gather(frames_hbm, out_hbm, buf0, buf1, si0, si1, so0, so1):
    wid = lax.axis_index("s") * _NC + lax.axis_index("c")
    b = wid // S
    s = wid % S
    src_t = (s * (T - 1)) // (S - 1)
    in0 = pltpu.async_copy(frames_hbm.at[b, src_t, 0], buf0, si0)
    in1 = pltpu.async_copy(frames_hbm.at[b, src_t, 1], buf1, si1)
    in0.wait()
    out0 = pltpu.async_copy(buf0, out_hbm.at[b, s, 0], so0)
    in1.wait()
    out1 = pltpu.async_copy(buf1, out_hbm.at[b, s, 1], so1)
    out0.wait()
    in2 = pltpu.async_copy(frames_hbm.at[b, src_t, 2], buf0, si0)
    in2.wait()
    out2 = pltpu.async_copy(buf0, out_hbm.at[b, s, 2], so0)
    out1.wait()
    out2.wait()


def _fast_copy_body(x_ref, o_ref):
    o_ref[...] = x_ref[...]


_TBLK = 16


def _fast_copy(frames):
    return pl.pallas_call(
        _fast_copy_body,
        grid=(B, T // _TBLK),
        in_specs=[pl.BlockSpec((1, _TBLK, C, H, W), lambda i, j: (i, j, 0, 0, 0))],
        out_specs=pl.BlockSpec((1, _TBLK, C, H, W), lambda i, j: (i, j, 0, 0, 0)),
        out_shape=jax.ShapeDtypeStruct((B, T, C, H, W), jnp.float32),
    )(frames)


def kernel(frames):
    return (_slow_gather(frames), _fast_copy(frames))
